# baseline (speedup 1.0000x reference)
import functools
import math

import jax
import jax.numpy as jnp
from jax.experimental import pallas as pl
from jax.experimental.pallas import tpu as pltpu

SEQ = 2048
HIDDEN = 1024
NUM_HEADS = 16
HEAD_DIM = 64
NUM_EXPERTS = 8
TOP_K = 2
FFN = 4096
CAP_FACTOR = 1.25
CAPACITY = int(math.ceil(SEQ * TOP_K / NUM_EXPERTS * CAP_FACTOR))
SPAD = NUM_EXPERTS * CAPACITY

SB = 256
RB = 512
FB = 512
DB = 512
CB = 512

_BF = jnp.bfloat16
_F32 = jnp.float32


def _dot(a, b):
    return jax.lax.dot_general(a, b, (((a.ndim - 1,), (0,)), ((), ())),
                               preferred_element_type=_F32)


def _k1_body(x_ref, w_ref, b_ref, wt_ref, o_ref):
    x = x_ref[...]
    mu = jnp.mean(x, axis=1, keepdims=True)
    var = jnp.mean((x - mu) ** 2, axis=1, keepdims=True)
    ln = (x - mu) * jax.lax.rsqrt(var + 1e-5) * w_ref[...][None, :] + b_ref[...][None, :]
    o_ref[...] = _dot(ln.astype(_BF), wt_ref[...])


def _ln_qkv(x, w, b, wt_bf):
    return pl.pallas_call(
        _k1_body,
        grid=(SEQ // SB,),
        in_specs=[
            pl.BlockSpec((SB, HIDDEN), lambda i: (i, 0)),
            pl.BlockSpec((HIDDEN,), lambda i: (0,)),
            pl.BlockSpec((HIDDEN,), lambda i: (0,)),
            pl.BlockSpec((HIDDEN, 3 * HIDDEN), lambda i: (0, 0)),
        ],
        out_specs=pl.BlockSpec((SB, 3 * HIDDEN), lambda i: (i, 0)),
        out_shape=jax.ShapeDtypeStruct((SEQ, 3 * HIDDEN), jnp.float32),
    )(x, w, b, wt_bf)


KT = SEQ // 2


def _k2_lo_body(q_ref, k_ref, v_ref, o_ref):
    qb = pl.program_id(1)
    s = jax.lax.dot_general(q_ref[0], k_ref[0], (((1,), (1,)), ((), ())),
                            preferred_element_type=jnp.float32)
    s = s * (1.0 / math.sqrt(HEAD_DIM))
    row = jax.lax.broadcasted_iota(jnp.int32, (SB, KT), 0) + qb * SB
    col = jax.lax.broadcasted_iota(jnp.int32, (SB, KT), 1)
    s = jnp.where(col <= row, s, jnp.float32(-1e9))
    m0 = jnp.max(s, axis=1, keepdims=True)
    e0 = jnp.exp(s - m0)
    den = jnp.sum(e0, axis=1, keepdims=True)
    o_ref[0] = _dot(e0.astype(_BF), v_ref[0]) / den


def _k2_hi_body(q_ref, k_ref, v_ref, o_ref):
    qb = pl.program_id(1)
    q = q_ref[0]
    k = k_ref[0]
    v = v_ref[0]
    s = jax.lax.dot_general(q, k, (((1,), (1,)), ((), ())),
                            preferred_element_type=jnp.float32)
    s = s * (1.0 / math.sqrt(HEAD_DIM))
    s0 = s[:, :KT]
    s1 = s[:, KT:]
    row = jax.lax.broadcasted_iota(jnp.int32, (SB, KT), 0) + (KT + qb * SB)
    col = jax.lax.broadcasted_iota(jnp.int32, (SB, KT), 1) + KT
    s1 = jnp.where(col <= row, s1, jnp.float32(-1e9))
    m0 = jnp.max(s0, axis=1, keepdims=True)
    m = jnp.maximum(m0, jnp.max(s1, axis=1, keepdims=True))
    scale0 = jnp.exp(m0 - m)
    e0 = jnp.exp(s0 - m0)
    e1 = jnp.exp(s1 - m)
    acc = _dot(e0.astype(_BF), v[:KT]) * scale0 + _dot(e1.astype(_BF), v[KT:])
    den = jnp.sum(e0, axis=1, keepdims=True) * scale0 + jnp.sum(e1, axis=1, keepdims=True)
    o_ref[0] = acc / den


def _attention(q, k, v):
    lo = pl.pallas_call(
        _k2_lo_body,
        grid=(NUM_HEADS, KT // SB),
        in_specs=[
            pl.BlockSpec((1, SB, HEAD_DIM), lambda h, i: (h, i, 0)),
            pl.BlockSpec((1, KT, HEAD_DIM), lambda h, i: (h, 0, 0)),
            pl.BlockSpec((1, KT, HEAD_DIM), lambda h, i: (h, 0, 0)),
        ],
        out_specs=pl.BlockSpec((1, SB, HEAD_DIM), lambda h, i: (h, i, 0)),
        out_shape=jax.ShapeDtypeStruct((NUM_HEADS, KT, HEAD_DIM), jnp.float32),
    )(q[:, :KT], k[:, :KT], v[:, :KT])
    hi = pl.pallas_call(
        _k2_hi_body,
        grid=(NUM_HEADS, KT // SB),
        in_specs=[
            pl.BlockSpec((1, SB, HEAD_DIM), lambda h, i: (h, i, 0)),
            pl.BlockSpec((1, SEQ, HEAD_DIM), lambda h, i: (h, 0, 0)),
            pl.BlockSpec((1, SEQ, HEAD_DIM), lambda h, i: (h, 0, 0)),
        ],
        out_specs=pl.BlockSpec((1, SB, HEAD_DIM), lambda h, i: (h, i, 0)),
        out_shape=jax.ShapeDtypeStruct((NUM_HEADS, KT, HEAD_DIM), jnp.float32),
    )(q[:, KT:], k, v)
    return jnp.concatenate([lo, hi], axis=1)


def _k3_body(a_ref, hid_ref, pw_ref, w_ref, b_ref, rw_ref,
             h_ref, x2_ref, idx_ref, prb_ref):
    po = _dot(a_ref[...].astype(_BF), pw_ref[...])
    h = hid_ref[...] + po
    h_ref[...] = h
    mu = jnp.mean(h, axis=1, keepdims=True)
    var = jnp.mean((h - mu) ** 2, axis=1, keepdims=True)
    x2 = (h - mu) * jax.lax.rsqrt(var + 1e-5) * w_ref[...][None, :] + b_ref[...][None, :]
    x2_ref[...] = x2
    logits = _dot(x2.astype(_BF), rw_ref[...])
    col = jax.lax.broadcasted_iota(jnp.int32, (SB, 128), 1)
    valid = col < NUM_EXPERTS
    logits = jnp.where(valid, logits, jnp.float32(-1e30))
    m = jnp.max(logits, axis=1, keepdims=True)
    e = jnp.exp(logits - m)
    e = jnp.where(valid, e, 0.0)
    probs = e / jnp.sum(e, axis=1, keepdims=True)
    m1 = jnp.max(probs, axis=1, keepdims=True)
    i1 = jnp.min(jnp.where(probs == m1, col, 1 << 20), axis=1, keepdims=True)
    probs2 = jnp.where(col == i1, jnp.float32(-1.0), probs)
    m2 = jnp.max(probs2, axis=1, keepdims=True)
    i2 = jnp.min(jnp.where(probs2 == m2, col, 1 << 20), axis=1, keepdims=True)
    denom = m1 + m2
    p1 = m1 / denom
    p2 = m2 / denom
    idx_ref[...] = jnp.where(col == 0, i1, jnp.where(col == 1, i2, 0)).astype(jnp.int32)
    prb_ref[...] = jnp.where(col == 0, p1, jnp.where(col == 1, p2, 0.0))


def _proj_ln2_router(attn_sp, hidden, pwt_bf, w2, b2, rw_pad_bf):
    return pl.pallas_call(
        _k3_body,
        grid=(SEQ // SB,),
        in_specs=[
            pl.BlockSpec((SB, HIDDEN), lambda i: (i, 0)),
            pl.BlockSpec((SB, HIDDEN), lambda i: (i, 0)),
            pl.BlockSpec((HIDDEN, HIDDEN), lambda i: (0, 0)),
            pl.BlockSpec((HIDDEN,), lambda i: (0,)),
            pl.BlockSpec((HIDDEN,), lambda i: (0,)),
            pl.BlockSpec((HIDDEN, 128), lambda i: (0, 0)),
        ],
        out_specs=[
            pl.BlockSpec((SB, HIDDEN), lambda i: (i, 0)),
            pl.BlockSpec((SB, HIDDEN), lambda i: (i, 0)),
            pl.BlockSpec((SB, 128), lambda i: (i, 0)),
            pl.BlockSpec((SB, 128), lambda i: (i, 0)),
        ],
        out_shape=[
            jax.ShapeDtypeStruct((SEQ, HIDDEN), jnp.float32),
            jax.ShapeDtypeStruct((SEQ, HIDDEN), jnp.float32),
            jax.ShapeDtypeStruct((SEQ, 128), jnp.int32),
            jax.ShapeDtypeStruct((SEQ, 128), jnp.float32),
        ],
    )(attn_sp, hidden, pwt_bf, w2, b2, rw_pad_bf)


def _k4_body(e_ref, p_ref, slot_ref, peff_ref, cnt_ref, carry):
    b = pl.program_id(0)

    @pl.when(b == 0)
    def _():
        carry[...] = jnp.zeros_like(carry)

    e = e_ref[0, 0, :]
    p = p_ref[0, 0, :]
    col = jax.lax.broadcasted_iota(jnp.int32, (RB, 128), 1)
    oh = (e[:, None] == col).astype(_BF)
    rowi = jax.lax.broadcasted_iota(jnp.int32, (RB, RB), 0)
    coli = jax.lax.broadcasted_iota(jnp.int32, (RB, RB), 1)
    tri = (rowi > coli).astype(_BF)
    cnt = _dot(tri, oh)
    rank = jnp.sum(oh.astype(_F32) * (cnt + carry[...]), axis=1)
    keep = rank < CAPACITY
    slot = e * CAPACITY + rank.astype(jnp.int32)
    slot_ref[0, 0, :] = jnp.where(keep, slot, SPAD)
    peff_ref[0, 0, :] = jnp.where(keep, p, 0.0)
    carry[...] += jnp.sum(oh.astype(_F32), axis=0, keepdims=True)

    @pl.when(b == pl.num_programs(0) - 1)
    def _():
        cnt_ref[...] = jnp.minimum(carry[...], float(CAPACITY)).astype(jnp.int32)


def _routing(eids3, probs3):
    nblk = SEQ * TOP_K // RB
    return pl.pallas_call(
        _k4_body,
        grid=(nblk,),
        in_specs=[
            pl.BlockSpec((1, 1, RB), lambda i: (i, 0, 0)),
            pl.BlockSpec((1, 1, RB), lambda i: (i, 0, 0)),
        ],
        out_specs=[
            pl.BlockSpec((1, 1, RB), lambda i: (i, 0, 0)),
            pl.BlockSpec((1, 1, RB), lambda i: (i, 0, 0)),
            pl.BlockSpec((1, 128), lambda i: (0, 0)),
        ],
        out_shape=[
            jax.ShapeDtypeStruct((nblk, 1, RB), jnp.int32),
            jax.ShapeDtypeStruct((nblk, 1, RB), jnp.float32),
            jax.ShapeDtypeStruct((1, 128), jnp.int32),
        ],
        scratch_shapes=[pltpu.VMEM((1, 128), jnp.float32)],
    )(eids3, probs3)


def _k6_body(s0_ref, s1_ref, x_ref, o_ref):
    r = pl.program_id(0)
    rows = jax.lax.broadcasted_iota(jnp.int32, (DB, SEQ), 0) + r * DB
    d = jnp.logical_or(rows == s0_ref[...], rows == s1_ref[...]).astype(_BF)
    o_ref[...] = _dot(d, x_ref[...].astype(_BF))


def _dispatch(slots0, slots1, x2):
    return pl.pallas_call(
        _k6_body,
        grid=(SPAD // DB,),
        in_specs=[
            pl.BlockSpec((1, SEQ), lambda i: (0, 0)),
            pl.BlockSpec((1, SEQ), lambda i: (0, 0)),
            pl.BlockSpec((SEQ, HIDDEN), lambda i: (0, 0)),
        ],
        out_specs=pl.BlockSpec((DB, HIDDEN), lambda i: (i, 0)),
        out_shape=jax.ShapeDtypeStruct((SPAD, HIDDEN), jnp.float32),
    )(slots0, slots1, x2)


def _k5_body(cnt_ref, x_ref, w1_ref, w2_ref, o_ref):
    e = pl.program_id(0)
    f = pl.program_id(1)
    rows = jax.lax.broadcasted_iota(jnp.int32, (CAPACITY, HIDDEN), 0)
    x = jnp.where(rows < cnt_ref[e], x_ref[...], 0.0)
    fc1 = _dot(x.astype(_BF), w1_ref[0])
    act = jax.nn.gelu(fc1)
    contrib = _dot(act.astype(_BF), w2_ref[0])

    @pl.when(f == 0)
    def _():
        o_ref[...] = contrib

    @pl.when(f != 0)
    def _():
        o_ref[...] += contrib


def _expert_ffn(counts, padded_x, w1_bf, w2_bf):
    grid_spec = pltpu.PrefetchScalarGridSpec(
        num_scalar_prefetch=1,
        grid=(NUM_EXPERTS, FFN // FB),
        in_specs=[
            pl.BlockSpec((CAPACITY, HIDDEN), lambda e, f, c: (e, 0)),
            pl.BlockSpec((1, HIDDEN, FB), lambda e, f, c: (e, 0, f)),
            pl.BlockSpec((1, FB, HIDDEN), lambda e, f, c: (e, f, 0)),
        ],
        out_specs=pl.BlockSpec((CAPACITY, HIDDEN), lambda e, f, c: (e, 0)),
    )
    return pl.pallas_call(
        _k5_body,
        grid_spec=grid_spec,
        out_shape=jax.ShapeDtypeStruct((SPAD, HIDDEN), jnp.float32),
    )(counts, padded_x, w1_bf, w2_bf)


def _k7_body(s0_ref, s1_ref, p0_ref, p1_ref, h_ref, eo_ref, o_ref):
    sb = pl.program_id(1)
    cols = jax.lax.broadcasted_iota(jnp.int32, (SB, CB), 1) + sb * CB
    s0 = s0_ref[0, 0, :][:, None]
    s1 = s1_ref[0, 0, :][:, None]
    p0 = p0_ref[0, 0, :][:, None]
    p1 = p1_ref[0, 0, :][:, None]
    g = (cols == s0).astype(jnp.float32) * p0 + (cols == s1).astype(jnp.float32) * p1
    contrib = _dot(g.astype(_BF), eo_ref[...].astype(_BF))

    @pl.when(sb == 0)
    def _():
        o_ref[...] = h_ref[...] + contrib

    @pl.when(sb != 0)
    def _():
        o_ref[...] += contrib


def _combine(slots0_3, slots1_3, p0_3, p1_3, h_attn, expert_out):
    return pl.pallas_call(
        _k7_body,
        grid=(SEQ // SB, SPAD // CB),
        in_specs=[
            pl.BlockSpec((1, 1, SB), lambda t, s: (t, 0, 0)),
            pl.BlockSpec((1, 1, SB), lambda t, s: (t, 0, 0)),
            pl.BlockSpec((1, 1, SB), lambda t, s: (t, 0, 0)),
            pl.BlockSpec((1, 1, SB), lambda t, s: (t, 0, 0)),
            pl.BlockSpec((SB, HIDDEN), lambda t, s: (t, 0)),
            pl.BlockSpec((CB, HIDDEN), lambda t, s: (s, 0)),
        ],
        out_specs=pl.BlockSpec((SB, HIDDEN), lambda t, s: (t, 0)),
        out_shape=jax.ShapeDtypeStruct((SEQ, HIDDEN), jnp.float32),
    )(slots0_3, slots1_3, p0_3, p1_3, h_attn, expert_out)


def kernel(hidden_states, ln1_weight, ln1_bias, ln2_weight, ln2_bias,
           qkv_weight, proj_weight, router_weight, moe_w1, moe_w2):
    S, B, H = hidden_states.shape
    x0 = hidden_states.reshape(S, H)

    qkv = _ln_qkv(x0, ln1_weight, ln1_bias, qkv_weight.T.astype(_BF))
    qkv_h = qkv.reshape(S, NUM_HEADS, 3 * HEAD_DIM)
    q = jnp.transpose(qkv_h[:, :, :HEAD_DIM], (1, 0, 2)).astype(_BF)
    k = jnp.transpose(qkv_h[:, :, HEAD_DIM:2 * HEAD_DIM], (1, 0, 2)).astype(_BF)
    v = jnp.transpose(qkv_h[:, :, 2 * HEAD_DIM:], (1, 0, 2)).astype(_BF)
    attn = _attention(q, k, v)
    attn_sp = jnp.transpose(attn, (1, 0, 2)).reshape(S, H)

    rw_pad = jnp.zeros((H, 128), _BF).at[:, :NUM_EXPERTS].set(
        router_weight.astype(_BF))
    h_attn, x2, idx_out, prb_out = _proj_ln2_router(
        attn_sp, x0, proj_weight.T.astype(_BF), ln2_weight, ln2_bias, rw_pad)

    total = S * TOP_K
    eids = jnp.stack([idx_out[:, 0], idx_out[:, 1]], axis=1).reshape(total)
    eprb = jnp.stack([prb_out[:, 0], prb_out[:, 1]], axis=1).reshape(total)
    nblk = total // RB
    slots3, peff3, counts = _routing(eids.reshape(nblk, 1, RB),
                                     eprb.reshape(nblk, 1, RB))
    slots = slots3.reshape(S, TOP_K)
    peff = peff3.reshape(S, TOP_K)

    padded_x = _dispatch(slots[:, 0].reshape(1, S), slots[:, 1].reshape(1, S), x2)
    expert_out = _expert_ffn(counts[0, :NUM_EXPERTS], padded_x,
                             moe_w1.astype(_BF), moe_w2.astype(_BF))

    nb = SEQ // SB
    out = _combine(
        slots[:, 0].reshape(nb, 1, SB), slots[:, 1].reshape(nb, 1, SB),
        peff[:, 0].reshape(nb, 1, SB), peff[:, 1].reshape(nb, 1, SB),
        h_attn, expert_out)
    return out.reshape(S, B, H)


# speedup vs baseline: 1.1018x; 1.1018x over previous
import functools
import math

import jax
import jax.numpy as jnp
from jax import lax
from jax.experimental import pallas as pl
from jax.experimental.pallas import tpu as pltpu
from jax.experimental.pallas import tpu_sc as plsc

SEQ = 2048
HIDDEN = 1024
NUM_HEADS = 16
HEAD_DIM = 64
NUM_EXPERTS = 8
TOP_K = 2
FFN = 4096
CAP_FACTOR = 1.25
CAPACITY = int(math.ceil(SEQ * TOP_K / NUM_EXPERTS * CAP_FACTOR))
SPAD = NUM_EXPERTS * CAPACITY

SB = 256
RB = 512
FB = 512
DB = 512
CB = 512

_BF = jnp.bfloat16
_F32 = jnp.float32


def _dot(a, b):
    return jax.lax.dot_general(a, b, (((a.ndim - 1,), (0,)), ((), ())),
                               preferred_element_type=_F32)


def _k1_body(x_ref, w_ref, b_ref, wt_ref, o_ref):
    x = x_ref[...]
    mu = jnp.mean(x, axis=1, keepdims=True)
    var = jnp.mean((x - mu) ** 2, axis=1, keepdims=True)
    ln = (x - mu) * jax.lax.rsqrt(var + 1e-5) * w_ref[...][None, :] + b_ref[...][None, :]
    o_ref[...] = _dot(ln.astype(_BF), wt_ref[...])


def _ln_qkv(x, w, b, wt_bf):
    return pl.pallas_call(
        _k1_body,
        grid=(SEQ // SB,),
        in_specs=[
            pl.BlockSpec((SB, HIDDEN), lambda i: (i, 0)),
            pl.BlockSpec((HIDDEN,), lambda i: (0,)),
            pl.BlockSpec((HIDDEN,), lambda i: (0,)),
            pl.BlockSpec((HIDDEN, 3 * HIDDEN), lambda i: (0, 0)),
        ],
        out_specs=pl.BlockSpec((SB, 3 * HIDDEN), lambda i: (i, 0)),
        out_shape=jax.ShapeDtypeStruct((SEQ, 3 * HIDDEN), jnp.float32),
    )(x, w, b, wt_bf)


def _k2_body(q_ref, k_ref, v_ref, o_ref):
    qb = pl.program_id(1)
    q = q_ref[0]
    k = k_ref[0]
    v = v_ref[0]
    s = jax.lax.dot_general(q, k, (((1,), (1,)), ((), ())),
                            preferred_element_type=jnp.float32)
    s = s * (1.0 / math.sqrt(HEAD_DIM))
    row = jax.lax.broadcasted_iota(jnp.int32, (SB, SEQ), 0) + qb * SB
    col = jax.lax.broadcasted_iota(jnp.int32, (SB, SEQ), 1)
    s = jnp.where(col <= row, s, jnp.float32(-1e9))
    KT = SEQ // 2
    s0 = s[:, :KT]
    s1 = s[:, KT:]
    m0 = jnp.max(s0, axis=1, keepdims=True)
    m = jnp.maximum(m0, jnp.max(s1, axis=1, keepdims=True))
    scale0 = jnp.exp(m0 - m)
    e0 = jnp.exp(s0 - m0)
    e1 = jnp.exp(s1 - m)
    acc = _dot(e0.astype(_BF), v[:KT]) * scale0 + _dot(e1.astype(_BF), v[KT:])
    den = jnp.sum(e0, axis=1, keepdims=True) * scale0 + jnp.sum(e1, axis=1, keepdims=True)
    o_ref[0] = acc / den


def _attention(q, k, v):
    return pl.pallas_call(
        _k2_body,
        grid=(NUM_HEADS, SEQ // SB),
        in_specs=[
            pl.BlockSpec((1, SB, HEAD_DIM), lambda h, i: (h, i, 0)),
            pl.BlockSpec((1, SEQ, HEAD_DIM), lambda h, i: (h, 0, 0)),
            pl.BlockSpec((1, SEQ, HEAD_DIM), lambda h, i: (h, 0, 0)),
        ],
        out_specs=pl.BlockSpec((1, SB, HEAD_DIM), lambda h, i: (h, i, 0)),
        out_shape=jax.ShapeDtypeStruct((NUM_HEADS, SEQ, HEAD_DIM), jnp.float32),
    )(q, k, v)


def _k3_body(a_ref, hid_ref, pw_ref, w_ref, b_ref, rw_ref,
             h_ref, x2_ref, idx_ref, prb_ref):
    po = _dot(a_ref[...].astype(_BF), pw_ref[...])
    h = hid_ref[...] + po
    h_ref[...] = h
    mu = jnp.mean(h, axis=1, keepdims=True)
    var = jnp.mean((h - mu) ** 2, axis=1, keepdims=True)
    x2 = (h - mu) * jax.lax.rsqrt(var + 1e-5) * w_ref[...][None, :] + b_ref[...][None, :]
    x2_ref[...] = x2
    logits = _dot(x2.astype(_BF), rw_ref[...])
    col = jax.lax.broadcasted_iota(jnp.int32, (SB, 128), 1)
    valid = col < NUM_EXPERTS
    logits = jnp.where(valid, logits, jnp.float32(-1e30))
    m = jnp.max(logits, axis=1, keepdims=True)
    e = jnp.exp(logits - m)
    e = jnp.where(valid, e, 0.0)
    probs = e / jnp.sum(e, axis=1, keepdims=True)
    m1 = jnp.max(probs, axis=1, keepdims=True)
    i1 = jnp.min(jnp.where(probs == m1, col, 1 << 20), axis=1, keepdims=True)
    probs2 = jnp.where(col == i1, jnp.float32(-1.0), probs)
    m2 = jnp.max(probs2, axis=1, keepdims=True)
    i2 = jnp.min(jnp.where(probs2 == m2, col, 1 << 20), axis=1, keepdims=True)
    denom = m1 + m2
    p1 = m1 / denom
    p2 = m2 / denom
    idx_ref[...] = jnp.where(col == 0, i1, jnp.where(col == 1, i2, 0)).astype(jnp.int32)
    prb_ref[...] = jnp.where(col == 0, p1, jnp.where(col == 1, p2, 0.0))


def _proj_ln2_router(attn_sp, hidden, pwt_bf, w2, b2, rw_pad_bf):
    return pl.pallas_call(
        _k3_body,
        grid=(SEQ // SB,),
        in_specs=[
            pl.BlockSpec((SB, HIDDEN), lambda i: (i, 0)),
            pl.BlockSpec((SB, HIDDEN), lambda i: (i, 0)),
            pl.BlockSpec((HIDDEN, HIDDEN), lambda i: (0, 0)),
            pl.BlockSpec((HIDDEN,), lambda i: (0,)),
            pl.BlockSpec((HIDDEN,), lambda i: (0,)),
            pl.BlockSpec((HIDDEN, 128), lambda i: (0, 0)),
        ],
        out_specs=[
            pl.BlockSpec((SB, HIDDEN), lambda i: (i, 0)),
            pl.BlockSpec((SB, HIDDEN), lambda i: (i, 0)),
            pl.BlockSpec((SB, 128), lambda i: (i, 0)),
            pl.BlockSpec((SB, 128), lambda i: (i, 0)),
        ],
        out_shape=[
            jax.ShapeDtypeStruct((SEQ, HIDDEN), jnp.float32),
            jax.ShapeDtypeStruct((SEQ, HIDDEN), jnp.float32),
            jax.ShapeDtypeStruct((SEQ, 128), jnp.int32),
            jax.ShapeDtypeStruct((SEQ, 128), jnp.float32),
        ],
    )(attn_sp, hidden, pwt_bf, w2, b2, rw_pad_bf)


def _k4_body(e_ref, p_ref, slot_ref, cslot_ref, peb0_ref, peb1_ref, cnt_ref, carry):
    b = pl.program_id(0)

    @pl.when(b == 0)
    def _():
        carry[...] = jnp.zeros_like(carry)

    e = e_ref[0, 0, :]
    p = p_ref[0, 0, :]
    col = jax.lax.broadcasted_iota(jnp.int32, (RB, 128), 1)
    oh = (e[:, None] == col).astype(_BF)
    rowi = jax.lax.broadcasted_iota(jnp.int32, (RB, RB), 0)
    coli = jax.lax.broadcasted_iota(jnp.int32, (RB, RB), 1)
    tri = (rowi > coli).astype(_BF)
    cnt = _dot(tri, oh)
    rank = jnp.sum(oh.astype(_F32) * (cnt + carry[...]), axis=1)
    keep = rank < CAPACITY
    slot = e * CAPACITY + rank.astype(jnp.int32)
    slot_ref[0, 0, :] = jnp.where(keep, slot, SPAD)
    cslot_ref[0, 0, :] = jnp.where(keep, slot, 0)
    peff = jnp.where(keep, p, 0.0)
    pe2 = peff.reshape(RB // TOP_K, TOP_K)
    peb0_ref[...] = jnp.broadcast_to(pe2[:, 0:1], (RB // TOP_K, 16))
    peb1_ref[...] = jnp.broadcast_to(pe2[:, 1:2], (RB // TOP_K, 16))
    carry[...] += jnp.sum(oh.astype(_F32), axis=0, keepdims=True)

    @pl.when(b == pl.num_programs(0) - 1)
    def _():
        cnt_ref[...] = jnp.minimum(carry[...], float(CAPACITY)).astype(jnp.int32)


def _routing(eids3, probs3):
    nblk = SEQ * TOP_K // RB
    return pl.pallas_call(
        _k4_body,
        grid=(nblk,),
        in_specs=[
            pl.BlockSpec((1, 1, RB), lambda i: (i, 0, 0)),
            pl.BlockSpec((1, 1, RB), lambda i: (i, 0, 0)),
        ],
        out_specs=[
            pl.BlockSpec((1, 1, RB), lambda i: (i, 0, 0)),
            pl.BlockSpec((1, 1, RB), lambda i: (i, 0, 0)),
            pl.BlockSpec((RB // TOP_K, 16), lambda i: (i, 0)),
            pl.BlockSpec((RB // TOP_K, 16), lambda i: (i, 0)),
            pl.BlockSpec((1, 128), lambda i: (0, 0)),
        ],
        out_shape=[
            jax.ShapeDtypeStruct((nblk, 1, RB), jnp.int32),
            jax.ShapeDtypeStruct((nblk, 1, RB), jnp.int32),
            jax.ShapeDtypeStruct((SEQ, 16), jnp.float32),
            jax.ShapeDtypeStruct((SEQ, 16), jnp.float32),
            jax.ShapeDtypeStruct((1, 128), jnp.int32),
        ],
        scratch_shapes=[pltpu.VMEM((1, 128), jnp.float32)],
    )(eids3, probs3)


def _k6_body(s0_ref, s1_ref, x_ref, o_ref):
    r = pl.program_id(0)
    rows = jax.lax.broadcasted_iota(jnp.int32, (DB, SEQ), 0) + r * DB
    d = jnp.logical_or(rows == s0_ref[...], rows == s1_ref[...]).astype(_BF)
    o_ref[...] = _dot(d, x_ref[...].astype(_BF))


def _dispatch(slots0, slots1, x2):
    return pl.pallas_call(
        _k6_body,
        grid=(SPAD // DB,),
        in_specs=[
            pl.BlockSpec((1, SEQ), lambda i: (0, 0)),
            pl.BlockSpec((1, SEQ), lambda i: (0, 0)),
            pl.BlockSpec((SEQ, HIDDEN), lambda i: (0, 0)),
        ],
        out_specs=pl.BlockSpec((DB, HIDDEN), lambda i: (i, 0)),
        out_shape=jax.ShapeDtypeStruct((SPAD, HIDDEN), jnp.float32),
    )(slots0, slots1, x2)


def _k5_body(cnt_ref, x_ref, w1_ref, w2_ref, o_ref):
    e = pl.program_id(0)
    f = pl.program_id(1)
    rows = jax.lax.broadcasted_iota(jnp.int32, (CAPACITY, HIDDEN), 0)
    x = jnp.where(rows < cnt_ref[e], x_ref[...], 0.0)
    fc1 = _dot(x.astype(_BF), w1_ref[0])
    act = jax.nn.gelu(fc1)
    contrib = _dot(act.astype(_BF), w2_ref[0])

    @pl.when(f == 0)
    def _():
        o_ref[...] = contrib

    @pl.when(f != 0)
    def _():
        o_ref[...] += contrib


def _expert_ffn(counts, padded_x, w1_bf, w2_bf):
    grid_spec = pltpu.PrefetchScalarGridSpec(
        num_scalar_prefetch=1,
        grid=(NUM_EXPERTS, FFN // FB),
        in_specs=[
            pl.BlockSpec((CAPACITY, HIDDEN), lambda e, f, c: (e, 0)),
            pl.BlockSpec((1, HIDDEN, FB), lambda e, f, c: (e, 0, f)),
            pl.BlockSpec((1, FB, HIDDEN), lambda e, f, c: (e, f, 0)),
        ],
        out_specs=pl.BlockSpec((CAPACITY, HIDDEN), lambda e, f, c: (e, 0)),
    )
    return pl.pallas_call(
        _k5_body,
        grid_spec=grid_spec,
        out_shape=jax.ShapeDtypeStruct((SPAD, HIDDEN), jnp.float32),
    )(counts, padded_x, w1_bf, w2_bf)


def _k7_body(s0_ref, s1_ref, p0_ref, p1_ref, h_ref, eo_ref, o_ref):
    sb = pl.program_id(1)
    cols = jax.lax.broadcasted_iota(jnp.int32, (SB, CB), 1) + sb * CB
    s0 = s0_ref[0, 0, :][:, None]
    s1 = s1_ref[0, 0, :][:, None]
    p0 = p0_ref[0, 0, :][:, None]
    p1 = p1_ref[0, 0, :][:, None]
    g = (cols == s0).astype(jnp.float32) * p0 + (cols == s1).astype(jnp.float32) * p1
    contrib = _dot(g.astype(_BF), eo_ref[...].astype(_BF))

    @pl.when(sb == 0)
    def _():
        o_ref[...] = h_ref[...] + contrib

    @pl.when(sb != 0)
    def _():
        o_ref[...] += contrib


def _combine(slots0_3, slots1_3, p0_3, p1_3, h_attn, expert_out):
    return pl.pallas_call(
        _k7_body,
        grid=(SEQ // SB, SPAD // CB),
        in_specs=[
            pl.BlockSpec((1, 1, SB), lambda t, s: (t, 0, 0)),
            pl.BlockSpec((1, 1, SB), lambda t, s: (t, 0, 0)),
            pl.BlockSpec((1, 1, SB), lambda t, s: (t, 0, 0)),
            pl.BlockSpec((1, 1, SB), lambda t, s: (t, 0, 0)),
            pl.BlockSpec((SB, HIDDEN), lambda t, s: (t, 0)),
            pl.BlockSpec((CB, HIDDEN), lambda t, s: (s, 0)),
        ],
        out_specs=pl.BlockSpec((SB, HIDDEN), lambda t, s: (t, 0)),
        out_shape=jax.ShapeDtypeStruct((SEQ, HIDDEN), jnp.float32),
    )(slots0_3, slots1_3, p0_3, p1_3, h_attn, expert_out)


_SC_MESH = plsc.VectorSubcoreMesh(core_axis_name="c", subcore_axis_name="s")
_NW = 32
_TPW = SEQ // _NW
_L = 16


def _wid():
    return lax.axis_index("s") * 2 + lax.axis_index("c")


def _sc_dispatch(x2, ds0, ds1):
    @functools.partial(
        pl.kernel, mesh=_SC_MESH,
        out_type=jax.ShapeDtypeStruct((SPAD + 8, HIDDEN), jnp.float32),
        scratch_types=[pltpu.VMEM((_L, HIDDEN), jnp.float32),
                       pltpu.VMEM((_L,), jnp.int32),
                       pltpu.VMEM((_L,), jnp.int32),
                       pltpu.SemaphoreType.DMA])
    def k(x_hbm, ds0_hbm, ds1_hbm, out_hbm, xv, i0v, i1v, sem):
        base = _wid() * _TPW

        def body(c, _):
            off = base + c * _L
            pltpu.sync_copy(x_hbm.at[pl.ds(off, _L)], xv)
            pltpu.sync_copy(ds0_hbm.at[pl.ds(off, _L)], i0v)
            pltpu.sync_copy(ds1_hbm.at[pl.ds(off, _L)], i1v)
            pltpu.async_copy(xv, out_hbm.at[i0v], sem).wait()
            pltpu.async_copy(xv, out_hbm.at[i1v], sem).wait()
            return 0

        lax.fori_loop(0, _TPW // _L, body, 0)

    return k(x2, ds0, ds1)


def _sc_combine(h_attn, expert_out, cs0, cs1, peb0, peb1):
    @functools.partial(
        pl.kernel, mesh=_SC_MESH,
        out_type=jax.ShapeDtypeStruct((SEQ, HIDDEN), jnp.float32),
        scratch_types=[pltpu.VMEM((_L, HIDDEN), jnp.float32),
                       pltpu.VMEM((_L, HIDDEN), jnp.float32),
                       pltpu.VMEM((_L, HIDDEN), jnp.float32),
                       pltpu.VMEM((_L, HIDDEN), jnp.float32),
                       pltpu.VMEM((_L,), jnp.int32),
                       pltpu.VMEM((_L,), jnp.int32),
                       pltpu.VMEM((_L, _L), jnp.float32),
                       pltpu.VMEM((_L, _L), jnp.float32),
                       pltpu.SemaphoreType.DMA])
    def k(h_hbm, eo_hbm, cs0_hbm, cs1_hbm, pb0_hbm, pb1_hbm, out_hbm,
          g0, g1, hb, ob, i0v, i1v, q0v, q1v, sem):
        base = _wid() * _TPW

        def chunk(c, _):
            off = base + c * _L
            pltpu.sync_copy(cs0_hbm.at[pl.ds(off, _L)], i0v)
            pltpu.sync_copy(cs1_hbm.at[pl.ds(off, _L)], i1v)
            pltpu.sync_copy(pb0_hbm.at[pl.ds(off, _L)], q0v)
            pltpu.sync_copy(pb1_hbm.at[pl.ds(off, _L)], q1v)
            pltpu.sync_copy(h_hbm.at[pl.ds(off, _L)], hb)
            pltpu.async_copy(eo_hbm.at[i0v], g0, sem).wait()
            pltpu.async_copy(eo_hbm.at[i1v], g1, sem).wait()

            def cols(cc, _):
                csl = pl.ds(cc * _L, _L)
                for j in range(_L):
                    ob[j, csl] = (hb[j, csl] + q0v[j] * g0[j, csl]
                                  + q1v[j] * g1[j, csl])
                return 0

            lax.fori_loop(0, HIDDEN // _L, cols, 0)
            pltpu.sync_copy(ob, out_hbm.at[pl.ds(off, _L)])
            return 0

        lax.fori_loop(0, _TPW // _L, chunk, 0)

    return k(h_attn, expert_out, cs0, cs1, peb0, peb1)


def kernel(hidden_states, ln1_weight, ln1_bias, ln2_weight, ln2_bias,
           qkv_weight, proj_weight, router_weight, moe_w1, moe_w2):
    S, B, H = hidden_states.shape
    x0 = hidden_states.reshape(S, H)

    qkv = _ln_qkv(x0, ln1_weight, ln1_bias, qkv_weight.T.astype(_BF))
    qkv_h = qkv.reshape(S, NUM_HEADS, 3 * HEAD_DIM)
    q = jnp.transpose(qkv_h[:, :, :HEAD_DIM], (1, 0, 2)).astype(_BF)
    k = jnp.transpose(qkv_h[:, :, HEAD_DIM:2 * HEAD_DIM], (1, 0, 2)).astype(_BF)
    v = jnp.transpose(qkv_h[:, :, 2 * HEAD_DIM:], (1, 0, 2)).astype(_BF)
    attn = _attention(q, k, v)
    attn_sp = jnp.transpose(attn, (1, 0, 2)).reshape(S, H)

    rw_pad = jnp.zeros((H, 128), _BF).at[:, :NUM_EXPERTS].set(
        router_weight.astype(_BF))
    h_attn, x2, idx_out, prb_out = _proj_ln2_router(
        attn_sp, x0, proj_weight.T.astype(_BF), ln2_weight, ln2_bias, rw_pad)

    total = S * TOP_K
    eids = jnp.stack([idx_out[:, 0], idx_out[:, 1]], axis=1).reshape(total)
    eprb = jnp.stack([prb_out[:, 0], prb_out[:, 1]], axis=1).reshape(total)
    nblk = total // RB
    dslot3, cslot3, peb0, peb1, counts = _routing(eids.reshape(nblk, 1, RB),
                                                  eprb.reshape(nblk, 1, RB))
    dslots = dslot3.reshape(S, TOP_K)
    cslots = cslot3.reshape(S, TOP_K)

    padded_x = _sc_dispatch(x2, dslots[:, 0], dslots[:, 1])
    expert_out = _expert_ffn(counts[0, :NUM_EXPERTS], padded_x,
                             moe_w1.astype(_BF), moe_w2.astype(_BF))
    out = _sc_combine(h_attn, expert_out, cslots[:, 0], cslots[:, 1],
                      peb0, peb1)
    return out.reshape(S, B, H)


# speedup vs baseline: 1.1770x; 1.0682x over previous
import functools
import math

import jax
import jax.numpy as jnp
from jax import lax
from jax.experimental import pallas as pl
from jax.experimental.pallas import tpu as pltpu
from jax.experimental.pallas import tpu_sc as plsc

SEQ = 2048
HIDDEN = 1024
NUM_HEADS = 16
HEAD_DIM = 64
NUM_EXPERTS = 8
TOP_K = 2
FFN = 4096
CAP_FACTOR = 1.25
CAPACITY = int(math.ceil(SEQ * TOP_K / NUM_EXPERTS * CAP_FACTOR))
SPAD = NUM_EXPERTS * CAPACITY

SB = 256
RB = 512
FB = 1024
DB = 512
CB = 512

_BF = jnp.bfloat16
_F32 = jnp.float32


def _dot(a, b):
    return jax.lax.dot_general(a, b, (((a.ndim - 1,), (0,)), ((), ())),
                               preferred_element_type=_F32)


def _k1_body(x_ref, w_ref, b_ref, wt_ref, o_ref):
    x = x_ref[...]
    mu = jnp.mean(x, axis=1, keepdims=True)
    var = jnp.mean((x - mu) ** 2, axis=1, keepdims=True)
    ln = (x - mu) * jax.lax.rsqrt(var + 1e-5) * w_ref[...][None, :] + b_ref[...][None, :]
    o_ref[...] = _dot(ln.astype(_BF), wt_ref[...]).astype(_BF)


def _ln_qkv(x, w, b, wt_bf):
    return pl.pallas_call(
        _k1_body,
        grid=(SEQ // SB,),
        in_specs=[
            pl.BlockSpec((SB, HIDDEN), lambda i: (i, 0)),
            pl.BlockSpec((HIDDEN,), lambda i: (0,)),
            pl.BlockSpec((HIDDEN,), lambda i: (0,)),
            pl.BlockSpec((HIDDEN, 3 * HIDDEN), lambda i: (0, 0)),
        ],
        out_specs=pl.BlockSpec((SB, 3 * HIDDEN), lambda i: (i, 0)),
        out_shape=jax.ShapeDtypeStruct((SEQ, 3 * HIDDEN), jnp.bfloat16),
    )(x, w, b, wt_bf)


def _k2_body(q_ref, k_ref, v_ref, o_ref):
    qb = pl.program_id(1)
    q = q_ref[0]
    k = k_ref[0]
    v = v_ref[0]
    s = jax.lax.dot_general(q, k, (((1,), (1,)), ((), ())),
                            preferred_element_type=jnp.float32)
    s = s * (1.0 / math.sqrt(HEAD_DIM))
    row = jax.lax.broadcasted_iota(jnp.int32, (SB, SEQ), 0) + qb * SB
    col = jax.lax.broadcasted_iota(jnp.int32, (SB, SEQ), 1)
    s = jnp.where(col <= row, s, jnp.float32(-1e9))
    KT = SEQ // 2
    s0 = s[:, :KT]
    s1 = s[:, KT:]
    m0 = jnp.max(s0, axis=1, keepdims=True)
    m = jnp.maximum(m0, jnp.max(s1, axis=1, keepdims=True))
    scale0 = jnp.exp(m0 - m)
    e0 = jnp.exp(s0 - m0)
    e1 = jnp.exp(s1 - m)
    acc = _dot(e0.astype(_BF), v[:KT]) * scale0 + _dot(e1.astype(_BF), v[KT:])
    den = jnp.sum(e0, axis=1, keepdims=True) * scale0 + jnp.sum(e1, axis=1, keepdims=True)
    o_ref[0] = (acc / den).astype(_BF)


def _attention(q, k, v):
    return pl.pallas_call(
        _k2_body,
        grid=(NUM_HEADS, SEQ // SB),
        in_specs=[
            pl.BlockSpec((1, SB, HEAD_DIM), lambda h, i: (h, i, 0)),
            pl.BlockSpec((1, SEQ, HEAD_DIM), lambda h, i: (h, 0, 0)),
            pl.BlockSpec((1, SEQ, HEAD_DIM), lambda h, i: (h, 0, 0)),
        ],
        out_specs=pl.BlockSpec((1, SB, HEAD_DIM), lambda h, i: (h, i, 0)),
        out_shape=jax.ShapeDtypeStruct((NUM_HEADS, SEQ, HEAD_DIM), jnp.bfloat16),
    )(q, k, v)


def _k3_body(a_ref, hid_ref, pw_ref, w_ref, b_ref, rw_ref,
             h_ref, x2_ref, idx_ref, prb_ref):
    po = _dot(a_ref[...].astype(_BF), pw_ref[...])
    h = hid_ref[...] + po
    h_ref[...] = h
    mu = jnp.mean(h, axis=1, keepdims=True)
    var = jnp.mean((h - mu) ** 2, axis=1, keepdims=True)
    x2 = (h - mu) * jax.lax.rsqrt(var + 1e-5) * w_ref[...][None, :] + b_ref[...][None, :]
    x2_ref[...] = x2
    logits = _dot(x2.astype(_BF), rw_ref[...])
    col = jax.lax.broadcasted_iota(jnp.int32, (SB, 128), 1)
    valid = col < NUM_EXPERTS
    logits = jnp.where(valid, logits, jnp.float32(-1e30))
    m = jnp.max(logits, axis=1, keepdims=True)
    e = jnp.exp(logits - m)
    e = jnp.where(valid, e, 0.0)
    probs = e / jnp.sum(e, axis=1, keepdims=True)
    m1 = jnp.max(probs, axis=1, keepdims=True)
    i1 = jnp.min(jnp.where(probs == m1, col, 1 << 20), axis=1, keepdims=True)
    probs2 = jnp.where(col == i1, jnp.float32(-1.0), probs)
    m2 = jnp.max(probs2, axis=1, keepdims=True)
    i2 = jnp.min(jnp.where(probs2 == m2, col, 1 << 20), axis=1, keepdims=True)
    denom = m1 + m2
    p1 = m1 / denom
    p2 = m2 / denom
    idx_ref[...] = jnp.where(col == 0, i1, jnp.where(col == 1, i2, 0)).astype(jnp.int32)
    prb_ref[...] = jnp.where(col == 0, p1, jnp.where(col == 1, p2, 0.0))


def _proj_ln2_router(attn_sp, hidden, pwt_bf, w2, b2, rw_pad_bf):
    return pl.pallas_call(
        _k3_body,
        grid=(SEQ // SB,),
        in_specs=[
            pl.BlockSpec((SB, HIDDEN), lambda i: (i, 0)),
            pl.BlockSpec((SB, HIDDEN), lambda i: (i, 0)),
            pl.BlockSpec((HIDDEN, HIDDEN), lambda i: (0, 0)),
            pl.BlockSpec((HIDDEN,), lambda i: (0,)),
            pl.BlockSpec((HIDDEN,), lambda i: (0,)),
            pl.BlockSpec((HIDDEN, 128), lambda i: (0, 0)),
        ],
        out_specs=[
            pl.BlockSpec((SB, HIDDEN), lambda i: (i, 0)),
            pl.BlockSpec((SB, HIDDEN), lambda i: (i, 0)),
            pl.BlockSpec((SB, 128), lambda i: (i, 0)),
            pl.BlockSpec((SB, 128), lambda i: (i, 0)),
        ],
        out_shape=[
            jax.ShapeDtypeStruct((SEQ, HIDDEN), jnp.float32),
            jax.ShapeDtypeStruct((SEQ, HIDDEN), jnp.float32),
            jax.ShapeDtypeStruct((SEQ, 128), jnp.int32),
            jax.ShapeDtypeStruct((SEQ, 128), jnp.float32),
        ],
    )(attn_sp, hidden, pwt_bf, w2, b2, rw_pad_bf)


def _k4_body(e_ref, p_ref, slot_ref, cslot_ref, peb0_ref, peb1_ref, cnt_ref, carry):
    b = pl.program_id(0)

    @pl.when(b == 0)
    def _():
        carry[...] = jnp.zeros_like(carry)

    e = e_ref[0, 0, :]
    p = p_ref[0, 0, :]
    col = jax.lax.broadcasted_iota(jnp.int32, (RB, 128), 1)
    oh = (e[:, None] == col).astype(_BF)
    rowi = jax.lax.broadcasted_iota(jnp.int32, (RB, RB), 0)
    coli = jax.lax.broadcasted_iota(jnp.int32, (RB, RB), 1)
    tri = (rowi > coli).astype(_BF)
    cnt = _dot(tri, oh)
    rank = jnp.sum(oh.astype(_F32) * (cnt + carry[...]), axis=1)
    keep = rank < CAPACITY
    slot = e * CAPACITY + rank.astype(jnp.int32)
    slot_ref[0, 0, :] = jnp.where(keep, slot, SPAD)
    cslot_ref[0, 0, :] = jnp.where(keep, slot, 0)
    peff = jnp.where(keep, p, 0.0)
    pe2 = peff.reshape(RB // TOP_K, TOP_K)
    peb0_ref[...] = jnp.broadcast_to(pe2[:, 0:1], (RB // TOP_K, 16))
    peb1_ref[...] = jnp.broadcast_to(pe2[:, 1:2], (RB // TOP_K, 16))
    carry[...] += jnp.sum(oh.astype(_F32), axis=0, keepdims=True)

    @pl.when(b == pl.num_programs(0) - 1)
    def _():
        cnt_ref[...] = jnp.minimum(carry[...], float(CAPACITY)).astype(jnp.int32)


def _routing(eids3, probs3):
    nblk = SEQ * TOP_K // RB
    return pl.pallas_call(
        _k4_body,
        grid=(nblk,),
        in_specs=[
            pl.BlockSpec((1, 1, RB), lambda i: (i, 0, 0)),
            pl.BlockSpec((1, 1, RB), lambda i: (i, 0, 0)),
        ],
        out_specs=[
            pl.BlockSpec((1, 1, RB), lambda i: (i, 0, 0)),
            pl.BlockSpec((1, 1, RB), lambda i: (i, 0, 0)),
            pl.BlockSpec((RB // TOP_K, 16), lambda i: (i, 0)),
            pl.BlockSpec((RB // TOP_K, 16), lambda i: (i, 0)),
            pl.BlockSpec((1, 128), lambda i: (0, 0)),
        ],
        out_shape=[
            jax.ShapeDtypeStruct((nblk, 1, RB), jnp.int32),
            jax.ShapeDtypeStruct((nblk, 1, RB), jnp.int32),
            jax.ShapeDtypeStruct((SEQ, 16), jnp.float32),
            jax.ShapeDtypeStruct((SEQ, 16), jnp.float32),
            jax.ShapeDtypeStruct((1, 128), jnp.int32),
        ],
        scratch_shapes=[pltpu.VMEM((1, 128), jnp.float32)],
    )(eids3, probs3)


def _k6_body(s0_ref, s1_ref, x_ref, o_ref):
    r = pl.program_id(0)
    rows = jax.lax.broadcasted_iota(jnp.int32, (DB, SEQ), 0) + r * DB
    d = jnp.logical_or(rows == s0_ref[...], rows == s1_ref[...]).astype(_BF)
    o_ref[...] = _dot(d, x_ref[...].astype(_BF))


def _dispatch(slots0, slots1, x2):
    return pl.pallas_call(
        _k6_body,
        grid=(SPAD // DB,),
        in_specs=[
            pl.BlockSpec((1, SEQ), lambda i: (0, 0)),
            pl.BlockSpec((1, SEQ), lambda i: (0, 0)),
            pl.BlockSpec((SEQ, HIDDEN), lambda i: (0, 0)),
        ],
        out_specs=pl.BlockSpec((DB, HIDDEN), lambda i: (i, 0)),
        out_shape=jax.ShapeDtypeStruct((SPAD, HIDDEN), jnp.float32),
    )(slots0, slots1, x2)


def _k5_body(cnt_ref, x_ref, w1_ref, w2_ref, o_ref):
    e = pl.program_id(0)
    f = pl.program_id(1)
    rows = jax.lax.broadcasted_iota(jnp.int32, (CAPACITY, HIDDEN), 0)
    x = jnp.where(rows < cnt_ref[e], x_ref[...], 0.0)
    fc1 = _dot(x.astype(_BF), w1_ref[0])
    act = jax.nn.gelu(fc1)
    contrib = _dot(act.astype(_BF), w2_ref[0])

    @pl.when(f == 0)
    def _():
        o_ref[...] = contrib

    @pl.when(f != 0)
    def _():
        o_ref[...] += contrib


def _expert_ffn(counts, padded_x, w1_bf, w2_bf):
    grid_spec = pltpu.PrefetchScalarGridSpec(
        num_scalar_prefetch=1,
        grid=(NUM_EXPERTS, FFN // FB),
        in_specs=[
            pl.BlockSpec((CAPACITY, HIDDEN), lambda e, f, c: (e, 0)),
            pl.BlockSpec((1, HIDDEN, FB), lambda e, f, c: (e, 0, f)),
            pl.BlockSpec((1, FB, HIDDEN), lambda e, f, c: (e, f, 0)),
        ],
        out_specs=pl.BlockSpec((CAPACITY, HIDDEN), lambda e, f, c: (e, 0)),
    )
    return pl.pallas_call(
        _k5_body,
        grid_spec=grid_spec,
        out_shape=jax.ShapeDtypeStruct((SPAD, HIDDEN), jnp.float32),
    )(counts, padded_x, w1_bf, w2_bf)


def _k7_body(s0_ref, s1_ref, p0_ref, p1_ref, h_ref, eo_ref, o_ref):
    sb = pl.program_id(1)
    cols = jax.lax.broadcasted_iota(jnp.int32, (SB, CB), 1) + sb * CB
    s0 = s0_ref[0, 0, :][:, None]
    s1 = s1_ref[0, 0, :][:, None]
    p0 = p0_ref[0, 0, :][:, None]
    p1 = p1_ref[0, 0, :][:, None]
    g = (cols == s0).astype(jnp.float32) * p0 + (cols == s1).astype(jnp.float32) * p1
    contrib = _dot(g.astype(_BF), eo_ref[...].astype(_BF))

    @pl.when(sb == 0)
    def _():
        o_ref[...] = h_ref[...] + contrib

    @pl.when(sb != 0)
    def _():
        o_ref[...] += contrib


def _combine(slots0_3, slots1_3, p0_3, p1_3, h_attn, expert_out):
    return pl.pallas_call(
        _k7_body,
        grid=(SEQ // SB, SPAD // CB),
        in_specs=[
            pl.BlockSpec((1, 1, SB), lambda t, s: (t, 0, 0)),
            pl.BlockSpec((1, 1, SB), lambda t, s: (t, 0, 0)),
            pl.BlockSpec((1, 1, SB), lambda t, s: (t, 0, 0)),
            pl.BlockSpec((1, 1, SB), lambda t, s: (t, 0, 0)),
            pl.BlockSpec((SB, HIDDEN), lambda t, s: (t, 0)),
            pl.BlockSpec((CB, HIDDEN), lambda t, s: (s, 0)),
        ],
        out_specs=pl.BlockSpec((SB, HIDDEN), lambda t, s: (t, 0)),
        out_shape=jax.ShapeDtypeStruct((SEQ, HIDDEN), jnp.float32),
    )(slots0_3, slots1_3, p0_3, p1_3, h_attn, expert_out)


_SC_MESH = plsc.VectorSubcoreMesh(core_axis_name="c", subcore_axis_name="s")
_NW = 32
_TPW = SEQ // _NW
_L = 16


def _wid():
    return lax.axis_index("s") * 2 + lax.axis_index("c")


def _sc_dispatch(x2, ds0, ds1):
    @functools.partial(
        pl.kernel, mesh=_SC_MESH,
        out_type=jax.ShapeDtypeStruct((SPAD + 8, HIDDEN), jnp.float32),
        scratch_types=[pltpu.VMEM((_L, HIDDEN), jnp.float32),
                       pltpu.VMEM((_L,), jnp.int32),
                       pltpu.VMEM((_L,), jnp.int32),
                       pltpu.SemaphoreType.DMA])
    def k(x_hbm, ds0_hbm, ds1_hbm, out_hbm, xv, i0v, i1v, sem):
        base = _wid() * _TPW

        def body(c, _):
            off = base + c * _L
            pltpu.sync_copy(x_hbm.at[pl.ds(off, _L)], xv)
            pltpu.sync_copy(ds0_hbm.at[pl.ds(off, _L)], i0v)
            pltpu.sync_copy(ds1_hbm.at[pl.ds(off, _L)], i1v)
            pltpu.async_copy(xv, out_hbm.at[i0v], sem).wait()
            pltpu.async_copy(xv, out_hbm.at[i1v], sem).wait()
            return 0

        lax.fori_loop(0, _TPW // _L, body, 0)

    return k(x2, ds0, ds1)


def _sc_combine(h_attn, expert_out, cs0, cs1, peb0, peb1):
    @functools.partial(
        pl.kernel, mesh=_SC_MESH,
        out_type=jax.ShapeDtypeStruct((SEQ, HIDDEN), jnp.float32),
        scratch_types=[pltpu.VMEM((_L, HIDDEN), jnp.float32),
                       pltpu.VMEM((_L, HIDDEN), jnp.float32),
                       pltpu.VMEM((_L, HIDDEN), jnp.float32),
                       pltpu.VMEM((_L, HIDDEN), jnp.float32),
                       pltpu.VMEM((_L,), jnp.int32),
                       pltpu.VMEM((_L,), jnp.int32),
                       pltpu.VMEM((_L, _L), jnp.float32),
                       pltpu.VMEM((_L, _L), jnp.float32),
                       pltpu.SemaphoreType.DMA])
    def k(h_hbm, eo_hbm, cs0_hbm, cs1_hbm, pb0_hbm, pb1_hbm, out_hbm,
          g0, g1, hb, ob, i0v, i1v, q0v, q1v, sem):
        base = _wid() * _TPW

        def chunk(c, _):
            off = base + c * _L
            pltpu.sync_copy(cs0_hbm.at[pl.ds(off, _L)], i0v)
            pltpu.sync_copy(cs1_hbm.at[pl.ds(off, _L)], i1v)
            pltpu.sync_copy(pb0_hbm.at[pl.ds(off, _L)], q0v)
            pltpu.sync_copy(pb1_hbm.at[pl.ds(off, _L)], q1v)
            pltpu.sync_copy(h_hbm.at[pl.ds(off, _L)], hb)
            pltpu.async_copy(eo_hbm.at[i0v], g0, sem).wait()
            pltpu.async_copy(eo_hbm.at[i1v], g1, sem).wait()

            def cols(cc, _):
                csl = pl.ds(cc * _L, _L)
                for j in range(_L):
                    ob[j, csl] = (hb[j, csl] + q0v[j] * g0[j, csl]
                                  + q1v[j] * g1[j, csl])
                return 0

            lax.fori_loop(0, HIDDEN // _L, cols, 0)
            pltpu.sync_copy(ob, out_hbm.at[pl.ds(off, _L)])
            return 0

        lax.fori_loop(0, _TPW // _L, chunk, 0)

    return k(h_attn, expert_out, cs0, cs1, peb0, peb1)


def kernel(hidden_states, ln1_weight, ln1_bias, ln2_weight, ln2_bias,
           qkv_weight, proj_weight, router_weight, moe_w1, moe_w2):
    S, B, H = hidden_states.shape
    x0 = hidden_states.reshape(S, H)

    qkv = _ln_qkv(x0, ln1_weight, ln1_bias, qkv_weight.T.astype(_BF))
    qkv_h = qkv.reshape(S, NUM_HEADS, 3 * HEAD_DIM)
    q = jnp.transpose(qkv_h[:, :, :HEAD_DIM], (1, 0, 2)).astype(_BF)
    k = jnp.transpose(qkv_h[:, :, HEAD_DIM:2 * HEAD_DIM], (1, 0, 2)).astype(_BF)
    v = jnp.transpose(qkv_h[:, :, 2 * HEAD_DIM:], (1, 0, 2)).astype(_BF)
    attn = _attention(q, k, v)
    attn_sp = jnp.transpose(attn, (1, 0, 2)).reshape(S, H)

    rw_pad = jnp.zeros((H, 128), _BF).at[:, :NUM_EXPERTS].set(
        router_weight.astype(_BF))
    h_attn, x2, idx_out, prb_out = _proj_ln2_router(
        attn_sp, x0, proj_weight.T.astype(_BF), ln2_weight, ln2_bias, rw_pad)

    total = S * TOP_K
    eids = jnp.stack([idx_out[:, 0], idx_out[:, 1]], axis=1).reshape(total)
    eprb = jnp.stack([prb_out[:, 0], prb_out[:, 1]], axis=1).reshape(total)
    nblk = total // RB
    dslot3, cslot3, peb0, peb1, counts = _routing(eids.reshape(nblk, 1, RB),
                                                  eprb.reshape(nblk, 1, RB))
    dslots = dslot3.reshape(S, TOP_K)
    cslots = cslot3.reshape(S, TOP_K)

    padded_x = _sc_dispatch(x2, dslots[:, 0], dslots[:, 1])
    expert_out = _expert_ffn(counts[0, :NUM_EXPERTS], padded_x,
                             moe_w1.astype(_BF), moe_w2.astype(_BF))
    out = _sc_combine(h_attn, expert_out, cslots[:, 0], cslots[:, 1],
                      peb0, peb1)
    return out.reshape(S, B, H)
